# baseline (speedup 1.0000x reference)
import functools

import jax
import jax.numpy as jnp
from jax.experimental import pallas as pl
from jax.experimental.pallas import tpu as pltpu


def _router_kernel(x_ref, w_ref, b_ref, tri_ref,
                   routed_ref, rprobs_ref, idx_ref, carry_ref, *, capacity):
    i = pl.program_id(0)

    @pl.when(i == 0)
    def _():
        carry_ref[...] = jnp.zeros_like(carry_ref)

    logits = jax.lax.dot_general(
        x_ref[...], w_ref[...], (((1,), (0,)), ((), ())),
        preferred_element_type=jnp.float32) + b_ref[...]

    tb, ne = logits.shape
    lane = jax.lax.broadcasted_iota(jnp.int32, (tb, ne), 1)

    m1 = jnp.max(logits, axis=-1, keepdims=True)
    e = jnp.exp(logits - m1)
    rz = 1.0 / jnp.sum(e, axis=-1, keepdims=True)
    idx1 = jnp.argmax(e, axis=-1, keepdims=True)
    is1 = lane == idx1
    e2 = jnp.where(is1, -1.0, e)
    m2 = jnp.max(e2, axis=-1, keepdims=True)
    idx2 = jnp.argmax(e2, axis=-1, keepdims=True)
    is2 = lane == idx2

    oh = jnp.where(is1, 1.0, 0.0) + jnp.where(is2, 1.0, 0.0)
    inc = jax.lax.dot_general(
        tri_ref[...], oh.astype(jnp.bfloat16), (((1,), (0,)), ((), ())),
        preferred_element_type=jnp.float32)
    total = inc + carry_ref[...]
    carry_ref[...] = total[tb - 1:tb, :]

    routed = jnp.where(total <= capacity, oh, 0.0)
    routed_ref[...] = routed
    rprobs_ref[...] = routed * jnp.where(is1, rz, m2 * rz)

    col2 = jax.lax.broadcasted_iota(jnp.int32, (tb, 2), 1)
    idx_ref[...] = jnp.where(col2 == 0, idx1, idx2)


@jax.jit
def kernel(x, W, b):
    tokens, d = x.shape
    ne = W.shape[1]
    tb = 2048
    capacity = tokens / ne * 1.25
    tri = (jax.lax.broadcasted_iota(jnp.int32, (tb, tb), 0)
           >= jax.lax.broadcasted_iota(jnp.int32, (tb, tb), 1)
           ).astype(jnp.bfloat16)
    out_shape = (
        jax.ShapeDtypeStruct((tokens, ne), jnp.float32),
        jax.ShapeDtypeStruct((tokens, ne), jnp.float32),
        jax.ShapeDtypeStruct((tokens, 2), jnp.int32),
    )
    routed, rprobs, idx = pl.pallas_call(
        functools.partial(_router_kernel, capacity=capacity),
        grid=(tokens // tb,),
        in_specs=[
            pl.BlockSpec((tb, d), lambda i: (i, 0)),
            pl.BlockSpec((d, ne), lambda i: (0, 0)),
            pl.BlockSpec((1, ne), lambda i: (0, 0)),
            pl.BlockSpec((tb, tb), lambda i: (0, 0)),
        ],
        out_specs=(
            pl.BlockSpec((tb, ne), lambda i: (i, 0)),
            pl.BlockSpec((tb, ne), lambda i: (i, 0)),
            pl.BlockSpec((tb, 2), lambda i: (i, 0)),
        ),
        out_shape=out_shape,
        scratch_shapes=[pltpu.VMEM((1, ne), jnp.float32)],
        compiler_params=pltpu.CompilerParams(
            dimension_semantics=("arbitrary",)),
    )(x, W, b.reshape(1, ne), tri)
    return routed, rprobs, idx, 0.0


# speedup vs baseline: 1.0657x; 1.0657x over previous
import functools

import jax
import jax.numpy as jnp
from jax.experimental import pallas as pl
from jax.experimental.pallas import tpu as pltpu


def _router_kernel(x_ref, w_ref, b_ref, tri_ref,
                   routed_ref, rprobs_ref, idx_ref, carry_ref, *, capacity):
    i = pl.program_id(0)

    @pl.when(i == 0)
    def _():
        carry_ref[...] = jnp.zeros_like(carry_ref)

    logits = jax.lax.dot_general(
        x_ref[...], w_ref[...], (((1,), (0,)), ((), ())),
        preferred_element_type=jnp.float32) + b_ref[...]

    tb, ne = logits.shape
    lane = jax.lax.broadcasted_iota(jnp.int32, (tb, ne), 1)

    m1 = jnp.max(logits, axis=-1, keepdims=True)
    e = jnp.exp(logits - m1)
    rz = 1.0 / jnp.sum(e, axis=-1, keepdims=True)
    idx1 = jnp.argmax(e, axis=-1, keepdims=True)
    is1 = lane == idx1
    e2 = jnp.where(is1, -1.0, e)
    m2 = jnp.max(e2, axis=-1, keepdims=True)
    idx2 = jnp.argmax(e2, axis=-1, keepdims=True)
    is2 = lane == idx2

    oh = jnp.where(is1, 1.0, 0.0) + jnp.where(is2, 1.0, 0.0)
    inc = jax.lax.dot_general(
        tri_ref[...], oh.astype(jnp.bfloat16), (((1,), (0,)), ((), ())),
        preferred_element_type=jnp.float32)
    room = capacity - carry_ref[...]
    carry_ref[...] = carry_ref[...] + inc[tb - 1:tb, :]

    routed = jnp.where(inc <= room, oh, 0.0)
    routed_ref[...] = routed
    rprobs_ref[...] = routed * jnp.where(is1, rz, m2 * rz)

    col2 = jax.lax.broadcasted_iota(jnp.int32, (tb, 2), 1)
    idx_ref[...] = jnp.where(col2 == 0, idx1, idx2)


@jax.jit
def kernel(x, W, b):
    tokens, d = x.shape
    ne = W.shape[1]
    tb = 1024
    capacity = tokens / ne * 1.25
    tri = (jax.lax.broadcasted_iota(jnp.int32, (tb, tb), 0)
           >= jax.lax.broadcasted_iota(jnp.int32, (tb, tb), 1)
           ).astype(jnp.bfloat16)
    out_shape = (
        jax.ShapeDtypeStruct((tokens, ne), jnp.float32),
        jax.ShapeDtypeStruct((tokens, ne), jnp.float32),
        jax.ShapeDtypeStruct((tokens, 2), jnp.int32),
    )
    routed, rprobs, idx = pl.pallas_call(
        functools.partial(_router_kernel, capacity=capacity),
        grid=(tokens // tb,),
        in_specs=[
            pl.BlockSpec((tb, d), lambda i: (i, 0)),
            pl.BlockSpec((d, ne), lambda i: (0, 0)),
            pl.BlockSpec((1, ne), lambda i: (0, 0)),
            pl.BlockSpec((tb, tb), lambda i: (0, 0)),
        ],
        out_specs=(
            pl.BlockSpec((tb, ne), lambda i: (i, 0)),
            pl.BlockSpec((tb, ne), lambda i: (i, 0)),
            pl.BlockSpec((tb, 2), lambda i: (i, 0)),
        ),
        out_shape=out_shape,
        scratch_shapes=[pltpu.VMEM((1, ne), jnp.float32)],
        compiler_params=pltpu.CompilerParams(
            dimension_semantics=("arbitrary",)),
    )(x, W, b.reshape(1, ne), tri)
    return routed, rprobs, idx, 0.0


# speedup vs baseline: 1.1361x; 1.0660x over previous
import functools

import jax
import jax.numpy as jnp
from jax.experimental import pallas as pl
from jax.experimental.pallas import tpu as pltpu


def _router_kernel(x_ref, w_ref, b_ref, triu_ref,
                   routed_ref, rprobs_ref, idx_ref, carry_ref, *, capacity):
    i = pl.program_id(0)

    @pl.when(i == 0)
    def _():
        carry_ref[...] = jnp.zeros_like(carry_ref)

    logits = jax.lax.dot_general(
        x_ref[...], w_ref[...], (((1,), (0,)), ((), ())),
        preferred_element_type=jnp.float32) + b_ref[...]
    tb, ne = logits.shape
    lt = logits.T

    sub = jax.lax.broadcasted_iota(jnp.int32, (ne, 1), 0)
    m1 = jnp.max(lt, axis=0, keepdims=True)
    e = jnp.exp(lt - m1)
    rz = 1.0 / jnp.sum(e, axis=0, keepdims=True)
    idx1 = jnp.argmax(e, axis=0, keepdims=True)
    is1 = sub == idx1
    e2 = jnp.where(is1, -1.0, e)
    m2 = jnp.max(e2, axis=0, keepdims=True)
    idx2 = jnp.argmax(e2, axis=0, keepdims=True)
    is2 = sub == idx2

    oh = jnp.where(is1, 1.0, 0.0) + jnp.where(is2, 1.0, 0.0)
    inc = jax.lax.dot_general(
        oh.astype(jnp.bfloat16), triu_ref[...], (((1,), (0,)), ((), ())),
        preferred_element_type=jnp.float32)
    room = capacity - carry_ref[...]
    carry_ref[...] = carry_ref[...] + inc[:, tb - 1:tb]

    routed = jnp.where(inc <= room, oh, 0.0)
    routed_ref[...] = routed.T
    rprobs_ref[...] = (routed * jnp.where(is1, rz, m2 * rz)).T

    idx_ref[...] = jnp.concatenate([idx1, idx2], axis=0).T


@jax.jit
def kernel(x, W, b):
    tokens, d = x.shape
    ne = W.shape[1]
    tb = 1024
    capacity = tokens / ne * 1.25
    triu = (jax.lax.broadcasted_iota(jnp.int32, (tb, tb), 0)
            <= jax.lax.broadcasted_iota(jnp.int32, (tb, tb), 1)
            ).astype(jnp.bfloat16)
    out_shape = (
        jax.ShapeDtypeStruct((tokens, ne), jnp.float32),
        jax.ShapeDtypeStruct((tokens, ne), jnp.float32),
        jax.ShapeDtypeStruct((tokens, 2), jnp.int32),
    )
    routed, rprobs, idx = pl.pallas_call(
        functools.partial(_router_kernel, capacity=capacity),
        grid=(tokens // tb,),
        in_specs=[
            pl.BlockSpec((tb, d), lambda i: (i, 0)),
            pl.BlockSpec((d, ne), lambda i: (0, 0)),
            pl.BlockSpec((1, ne), lambda i: (0, 0)),
            pl.BlockSpec((tb, tb), lambda i: (0, 0)),
        ],
        out_specs=(
            pl.BlockSpec((tb, ne), lambda i: (i, 0)),
            pl.BlockSpec((tb, ne), lambda i: (i, 0)),
            pl.BlockSpec((tb, 2), lambda i: (i, 0)),
        ),
        out_shape=out_shape,
        scratch_shapes=[pltpu.VMEM((ne, 1), jnp.float32)],
        compiler_params=pltpu.CompilerParams(
            dimension_semantics=("arbitrary",)),
    )(x, W, b.reshape(1, ne), triu)
    return routed, rprobs, idx, 0.0


# speedup vs baseline: 1.1725x; 1.0321x over previous
import functools

import jax
import jax.numpy as jnp
from jax.experimental import pallas as pl
from jax.experimental.pallas import tpu as pltpu


def _router_kernel(x_ref, w_ref, b_ref, triu_ref,
                   routed_ref, rprobs_ref, idx_ref, carry_ref, *, capacity):
    i = pl.program_id(0)

    @pl.when(i == 0)
    def _():
        carry_ref[...] = jnp.zeros_like(carry_ref)

    logits = jax.lax.dot_general(
        x_ref[...], w_ref[...], (((1,), (0,)), ((), ())),
        preferred_element_type=jnp.float32) + b_ref[...]
    tb, ne = logits.shape
    lt = logits.T

    sub = jax.lax.broadcasted_iota(jnp.int32, (ne, 1), 0)
    m1 = jnp.max(lt, axis=0, keepdims=True)
    e = jnp.exp(lt - m1)
    rz = 1.0 / jnp.sum(e, axis=0, keepdims=True)
    idx1 = jnp.argmax(e, axis=0, keepdims=True)
    is1 = sub == idx1
    e2 = jnp.where(is1, -1.0, e)
    m2 = jnp.max(e2, axis=0, keepdims=True)
    idx2 = jnp.argmax(e2, axis=0, keepdims=True)
    is2 = sub == idx2

    oh = jnp.where(is1, 1.0, 0.0) + jnp.where(is2, 1.0, 0.0)
    ohb = oh.astype(jnp.bfloat16)
    cw = 128
    tri = triu_ref[...]
    pv = jnp.where(is1, rz, m2 * rz)
    room = capacity - carry_ref[...]
    routed_chunks = []
    rprobs_chunks = []
    prefix = jnp.zeros((ne, 1), jnp.float32)
    for c in range(tb // cw):
        oh_c = jax.lax.slice_in_dim(oh, c * cw, (c + 1) * cw, axis=1)
        inc_c = jax.lax.dot_general(
            jax.lax.slice_in_dim(ohb, c * cw, (c + 1) * cw, axis=1), tri,
            (((1,), (0,)), ((), ())),
            preferred_element_type=jnp.float32) + prefix
        prefix = jax.lax.slice_in_dim(inc_c, cw - 1, cw, axis=1)
        routed_c = jnp.where(inc_c <= room, oh_c, 0.0)
        routed_chunks.append(routed_c)
        rprobs_chunks.append(
            routed_c * jax.lax.slice_in_dim(pv, c * cw, (c + 1) * cw, axis=1))
    carry_ref[...] = carry_ref[...] + prefix
    routed_ref[...] = jnp.concatenate(routed_chunks, axis=1).T
    rprobs_ref[...] = jnp.concatenate(rprobs_chunks, axis=1).T

    idx_ref[...] = jnp.concatenate([idx1, idx2], axis=0).T


@jax.jit
def kernel(x, W, b):
    tokens, d = x.shape
    ne = W.shape[1]
    tb = 1024
    capacity = tokens / ne * 1.25
    cw = 128
    triu = (jax.lax.broadcasted_iota(jnp.int32, (cw, cw), 0)
            <= jax.lax.broadcasted_iota(jnp.int32, (cw, cw), 1)
            ).astype(jnp.bfloat16)
    out_shape = (
        jax.ShapeDtypeStruct((tokens, ne), jnp.float32),
        jax.ShapeDtypeStruct((tokens, ne), jnp.float32),
        jax.ShapeDtypeStruct((tokens, 2), jnp.int32),
    )
    routed, rprobs, idx = pl.pallas_call(
        functools.partial(_router_kernel, capacity=capacity),
        grid=(tokens // tb,),
        in_specs=[
            pl.BlockSpec((tb, d), lambda i: (i, 0)),
            pl.BlockSpec((d, ne), lambda i: (0, 0)),
            pl.BlockSpec((1, ne), lambda i: (0, 0)),
            pl.BlockSpec((128, 128), lambda i: (0, 0)),
        ],
        out_specs=(
            pl.BlockSpec((tb, ne), lambda i: (i, 0)),
            pl.BlockSpec((tb, ne), lambda i: (i, 0)),
            pl.BlockSpec((tb, 2), lambda i: (i, 0)),
        ),
        out_shape=out_shape,
        scratch_shapes=[pltpu.VMEM((ne, 1), jnp.float32)],
        compiler_params=pltpu.CompilerParams(
            dimension_semantics=("arbitrary",)),
    )(x, W, b.reshape(1, ne), triu)
    return routed, rprobs, idx, 0.0


# speedup vs baseline: 2.0793x; 1.7734x over previous
import functools

import jax
import jax.numpy as jnp
from jax.experimental import pallas as pl
from jax.experimental.pallas import tpu as pltpu


def _router_kernel(x_ref, w_ref, b_ref, triu_ref,
                   routed_ref, rprobs_ref, idx_ref, carry_ref, *, capacity):
    i = pl.program_id(0)

    @pl.when(i == 0)
    def _():
        carry_ref[...] = jnp.zeros_like(carry_ref)

    logits = jax.lax.dot_general(
        x_ref[...], w_ref[...], (((1,), (0,)), ((), ())),
        preferred_element_type=jnp.float32) + b_ref[...]
    tb, ne = logits.shape
    lt = logits.T

    sub = jax.lax.broadcasted_iota(jnp.int32, (ne, 1), 0)
    m1 = jnp.max(lt, axis=0, keepdims=True)
    e = jnp.exp(lt - m1)
    rz = 1.0 / jnp.sum(e, axis=0, keepdims=True)
    idx1 = jnp.argmax(e, axis=0, keepdims=True)
    is1 = sub == idx1
    e2 = jnp.where(is1, -1.0, e)
    m2 = jnp.max(e2, axis=0, keepdims=True)
    idx2 = jnp.argmax(e2, axis=0, keepdims=True)
    is2 = sub == idx2

    oh = jnp.where(is1, 1.0, 0.0) + jnp.where(is2, 1.0, 0.0)
    ohb = oh.astype(jnp.bfloat16)
    cw = 128
    tri = triu_ref[...]
    pv = jnp.where(is1, rz, m2 * rz)
    room = capacity - carry_ref[...]
    routed_chunks = []
    rprobs_chunks = []
    prefix = jnp.zeros((ne, 1), jnp.float32)
    for c in range(tb // cw):
        oh_c = jax.lax.slice_in_dim(oh, c * cw, (c + 1) * cw, axis=1)
        inc_c = jax.lax.dot_general(
            jax.lax.slice_in_dim(ohb, c * cw, (c + 1) * cw, axis=1), tri,
            (((1,), (0,)), ((), ())),
            preferred_element_type=jnp.float32) + prefix
        prefix = jax.lax.slice_in_dim(inc_c, cw - 1, cw, axis=1)
        routed_c = jnp.where(inc_c <= room, oh_c, 0.0)
        routed_chunks.append(routed_c)
        rprobs_chunks.append(
            routed_c * jax.lax.slice_in_dim(pv, c * cw, (c + 1) * cw, axis=1))
    carry_ref[...] = carry_ref[...] + prefix
    routed_ref[...] = jnp.concatenate(routed_chunks, axis=1)
    rprobs_ref[...] = jnp.concatenate(rprobs_chunks, axis=1)

    idx_ref[...] = jnp.concatenate([idx1, idx2], axis=0)


@jax.jit
def kernel(x, W, b):
    tokens, d = x.shape
    ne = W.shape[1]
    tb = 1024
    capacity = tokens / ne * 1.25
    cw = 128
    triu = (jax.lax.broadcasted_iota(jnp.int32, (cw, cw), 0)
            <= jax.lax.broadcasted_iota(jnp.int32, (cw, cw), 1)
            ).astype(jnp.bfloat16)
    out_shape = (
        jax.ShapeDtypeStruct((ne, tokens), jnp.float32),
        jax.ShapeDtypeStruct((ne, tokens), jnp.float32),
        jax.ShapeDtypeStruct((2, tokens), jnp.int32),
    )
    routed, rprobs, idx = pl.pallas_call(
        functools.partial(_router_kernel, capacity=capacity),
        grid=(tokens // tb,),
        in_specs=[
            pl.BlockSpec((tb, d), lambda i: (i, 0)),
            pl.BlockSpec((d, ne), lambda i: (0, 0)),
            pl.BlockSpec((1, ne), lambda i: (0, 0)),
            pl.BlockSpec((128, 128), lambda i: (0, 0)),
        ],
        out_specs=(
            pl.BlockSpec((ne, tb), lambda i: (0, i)),
            pl.BlockSpec((ne, tb), lambda i: (0, i)),
            pl.BlockSpec((2, tb), lambda i: (0, i)),
        ),
        out_shape=out_shape,
        scratch_shapes=[pltpu.VMEM((ne, 1), jnp.float32)],
        compiler_params=pltpu.CompilerParams(
            dimension_semantics=("arbitrary",)),
    )(x, W, b.reshape(1, ne), triu)
    return routed.T, rprobs.T, idx.T, 0.0
